# baseline (speedup 1.0000x reference)
import functools

import jax
import jax.numpy as jnp
from jax import lax
from jax.experimental import pallas as pl
from jax.experimental.pallas import tpu as pltpu, tpu_sc as plsc

NUM_EXPS = 64
NUM_PNS = 5
VOCAB = 64
TOTAL = NUM_PNS * NUM_EXPS
LANES = 16
PACK = 2 * NUM_EXPS

_mesh = plsc.VectorSubcoreMesh(core_axis_name="c", subcore_axis_name="s")


@functools.partial(
    pl.kernel,
    mesh=_mesh,
    out_type=[
        jax.ShapeDtypeStruct((1, TOTAL), jnp.float32),
        jax.ShapeDtypeStruct((1, TOTAL), jnp.float32),
        jax.ShapeDtypeStruct((1, TOTAL), jnp.float32),
    ],
    scratch_types=[
        pltpu.VMEM((1,), jnp.int32),
        pltpu.VMEM((LANES,), jnp.int32),
        pltpu.VMEM((1, PACK), jnp.float32),
        pltpu.VMEM((NUM_PNS, NUM_EXPS), jnp.float32),
        pltpu.VMEM((1, TOTAL), jnp.float32),
        pltpu.VMEM((1, TOTAL), jnp.float32),
        pltpu.VMEM((1, TOTAL), jnp.float32),
        pltpu.SemaphoreType.DMA,
    ],
)
def _sc_forward(idx_hbm, par_hbm, w_stem_hbm, w_pn_hbm,
                diff_hbm, coal_hbm, pen_hbm,
                idx_v, par_v, row_v, pn_v, diff_v, coal_v, pen_v, sem):
    is_w0 = (lax.axis_index("c") == 0) & (lax.axis_index("s") == 0)

    @pl.when(is_w0)
    def _():
        pltpu.sync_copy(idx_hbm, idx_v)
        pltpu.sync_copy(par_hbm, par_v)
        pltpu.async_copy(w_stem_hbm.at[idx_v], row_v, sem).wait()
        pltpu.sync_copy(w_pn_hbm, pn_v)
        one = jnp.full((LANES,), 1.0, dtype=jnp.float32)
        odd = par_v[...] != 0
        sig_row = []
        for j in range(NUM_EXPS // LANES):
            lo = row_v[0, pl.ds(j * LANES, LANES)]
            hi = row_v[0, pl.ds(NUM_EXPS + j * LANES, LANES)]
            s = jnp.where(odd, hi, lo)
            sig_row.append(one / (one + jnp.exp(-s)))
        for p in range(NUM_PNS):
            for j in range(NUM_EXPS // LANES):
                base = p * NUM_EXPS + j * LANES
                q = pn_v[p, pl.ds(j * LANES, LANES)]
                c = sig_row[j] + one / (one + jnp.exp(-q))
                coal_v[0, pl.ds(base, LANES)] = c
                diff_v[0, pl.ds(base, LANES)] = c + c - one
                pen_v[0, pl.ds(base, LANES)] = one - c
        pltpu.sync_copy(diff_v, diff_hbm)
        pltpu.sync_copy(coal_v, coal_hbm)
        pltpu.sync_copy(pen_v, pen_hbm)


def kernel(stem_index, W_stem, W_pn, W_dep_bias, dep):
    si = stem_index.astype(jnp.int32)
    idx = jnp.reshape(si // 2, (1,))
    par = jnp.broadcast_to(si % 2, (LANES,))
    w_packed = jnp.reshape(W_stem, (VOCAB // 2, PACK))
    diff, coal, pen = _sc_forward(idx, par, w_packed, W_pn)
    return (diff, coal, pen, coal)


# speedup vs baseline: 1.1256x; 1.1256x over previous
import functools

import jax
import jax.numpy as jnp
from jax import lax
from jax.experimental import pallas as pl
from jax.experimental.pallas import tpu as pltpu, tpu_sc as plsc

NUM_EXPS = 64
NUM_PNS = 5
VOCAB = 64
TOTAL = NUM_PNS * NUM_EXPS
LANES = 16
PACK = 2 * NUM_EXPS

_mesh = plsc.VectorSubcoreMesh(core_axis_name="c", subcore_axis_name="s",
                               num_cores=1)


@functools.partial(
    pl.kernel,
    mesh=_mesh,
    out_type=[
        jax.ShapeDtypeStruct((1, TOTAL), jnp.float32),
        jax.ShapeDtypeStruct((1, TOTAL), jnp.float32),
        jax.ShapeDtypeStruct((1, TOTAL), jnp.float32),
    ],
    scratch_types=[
        pltpu.VMEM((1,), jnp.int32),
        pltpu.VMEM((LANES,), jnp.int32),
        pltpu.VMEM((1, PACK), jnp.float32),
        pltpu.VMEM((NUM_PNS, NUM_EXPS), jnp.float32),
        pltpu.VMEM((1, TOTAL), jnp.float32),
        pltpu.VMEM((1, TOTAL), jnp.float32),
        pltpu.VMEM((1, TOTAL), jnp.float32),
        pltpu.SemaphoreType.DMA,
        pltpu.SemaphoreType.DMA,
    ],
)
def _sc_forward(idx_hbm, par_hbm, w_stem_hbm, w_pn_hbm,
                diff_hbm, coal_hbm, pen_hbm,
                idx_v, par_v, row_v, pn_v, diff_v, coal_v, pen_v,
                sem_in, sem_out):
    is_w0 = (lax.axis_index("c") == 0) & (lax.axis_index("s") == 0)

    @pl.when(is_w0)
    def _():
        c_idx = pltpu.async_copy(idx_hbm, idx_v, sem_in)
        c_par = pltpu.async_copy(par_hbm, par_v, sem_in)
        c_pn = pltpu.async_copy(w_pn_hbm, pn_v, sem_in)
        c_idx.wait()
        pltpu.async_copy(w_stem_hbm.at[idx_v], row_v, sem_out).wait()
        c_par.wait()
        c_pn.wait()
        one = jnp.full((LANES,), 1.0, dtype=jnp.float32)
        odd = par_v[...] != 0
        sig_row = []
        for j in range(NUM_EXPS // LANES):
            lo = row_v[0, pl.ds(j * LANES, LANES)]
            hi = row_v[0, pl.ds(NUM_EXPS + j * LANES, LANES)]
            s = jnp.where(odd, hi, lo)
            sig_row.append(one / (one + jnp.exp(-s)))
        for p in range(NUM_PNS):
            for j in range(NUM_EXPS // LANES):
                base = p * NUM_EXPS + j * LANES
                q = pn_v[p, pl.ds(j * LANES, LANES)]
                c = sig_row[j] + one / (one + jnp.exp(-q))
                coal_v[0, pl.ds(base, LANES)] = c
                diff_v[0, pl.ds(base, LANES)] = c + c - one
                pen_v[0, pl.ds(base, LANES)] = one - c
        d1 = pltpu.async_copy(diff_v, diff_hbm, sem_out)
        d2 = pltpu.async_copy(coal_v, coal_hbm, sem_out)
        d3 = pltpu.async_copy(pen_v, pen_hbm, sem_out)
        d1.wait()
        d2.wait()
        d3.wait()


def kernel(stem_index, W_stem, W_pn, W_dep_bias, dep):
    si = stem_index.astype(jnp.int32)
    idx = jnp.reshape(si // 2, (1,))
    par = jnp.broadcast_to(si % 2, (LANES,))
    w_packed = jnp.reshape(W_stem, (VOCAB // 2, PACK))
    diff, coal, pen = _sc_forward(idx, par, w_packed, W_pn)
    return (diff, coal, pen, coal)


# speedup vs baseline: 1.1305x; 1.0044x over previous
import functools

import jax
import jax.numpy as jnp
from jax import lax
from jax.experimental import pallas as pl
from jax.experimental.pallas import tpu as pltpu, tpu_sc as plsc

NUM_EXPS = 64
NUM_PNS = 5
VOCAB = 64
TOTAL = NUM_PNS * NUM_EXPS
LANES = 16
PACK = 2 * NUM_EXPS

_mesh = plsc.VectorSubcoreMesh(core_axis_name="c", subcore_axis_name="s",
                               num_cores=1)


@functools.partial(
    pl.kernel,
    mesh=_mesh,
    out_type=[
        jax.ShapeDtypeStruct((1, TOTAL), jnp.float32),
        jax.ShapeDtypeStruct((1, TOTAL), jnp.float32),
        jax.ShapeDtypeStruct((1, TOTAL), jnp.float32),
    ],
    scratch_types=[
        pltpu.VMEM((1,), jnp.int32),
        pltpu.VMEM((LANES,), jnp.int32),
        pltpu.VMEM((1, PACK), jnp.float32),
        pltpu.VMEM((NUM_PNS, NUM_EXPS), jnp.float32),
        pltpu.VMEM((1, TOTAL), jnp.float32),
        pltpu.VMEM((1, TOTAL), jnp.float32),
        pltpu.VMEM((1, TOTAL), jnp.float32),
        pltpu.SemaphoreType.DMA,
        pltpu.SemaphoreType.DMA,
    ],
)
def _sc_forward(idx_hbm, par_hbm, w_stem_hbm, w_pn_hbm,
                diff_hbm, coal_hbm, pen_hbm,
                idx_v, par_v, row_v, pn_v, diff_v, coal_v, pen_v,
                sem_in, sem_out):
    is_w0 = (lax.axis_index("c") == 0) & (lax.axis_index("s") == 0)

    @pl.when(is_w0)
    def _():
        c_idx = pltpu.async_copy(idx_hbm, idx_v, sem_in)
        c_par = pltpu.async_copy(par_hbm, par_v, sem_in)
        c_pn = pltpu.async_copy(w_pn_hbm, pn_v, sem_in)
        c_idx.wait()
        gather = pltpu.async_copy(w_stem_hbm.at[idx_v], row_v, sem_out)
        one = jnp.full((LANES,), 1.0, dtype=jnp.float32)
        c_pn.wait()
        sig_pn = []
        for p in range(NUM_PNS):
            for j in range(NUM_EXPS // LANES):
                q = pn_v[p, pl.ds(j * LANES, LANES)]
                sig_pn.append(one / (one + jnp.exp(-q)))
        gather.wait()
        c_par.wait()
        odd = par_v[...] != 0
        sig_row = []
        for j in range(NUM_EXPS // LANES):
            lo = row_v[0, pl.ds(j * LANES, LANES)]
            hi = row_v[0, pl.ds(NUM_EXPS + j * LANES, LANES)]
            s = jnp.where(odd, hi, lo)
            sig_row.append(one / (one + jnp.exp(-s)))
        for p in range(NUM_PNS):
            for j in range(NUM_EXPS // LANES):
                base = p * NUM_EXPS + j * LANES
                c = sig_row[j] + sig_pn[p * (NUM_EXPS // LANES) + j]
                coal_v[0, pl.ds(base, LANES)] = c
                diff_v[0, pl.ds(base, LANES)] = c + c - one
                pen_v[0, pl.ds(base, LANES)] = one - c
        d1 = pltpu.async_copy(diff_v, diff_hbm, sem_out)
        d2 = pltpu.async_copy(coal_v, coal_hbm, sem_out)
        d3 = pltpu.async_copy(pen_v, pen_hbm, sem_out)
        d1.wait()
        d2.wait()
        d3.wait()


def kernel(stem_index, W_stem, W_pn, W_dep_bias, dep):
    si = stem_index.astype(jnp.int32)
    idx = jnp.reshape(si // 2, (1,))
    par = jnp.broadcast_to(si % 2, (LANES,))
    w_packed = jnp.reshape(W_stem, (VOCAB // 2, PACK))
    diff, coal, pen = _sc_forward(idx, par, w_packed, W_pn)
    return (diff, coal, pen, coal)
